# baseline (speedup 1.0000x reference)
import functools

import jax
import jax.numpy as jnp
from jax import lax
from jax.experimental import pallas as pl
from jax.experimental.pallas import tpu as pltpu
from jax.experimental.pallas import tpu_sc as plsc

B = 16384
D = 128
NC = 2
NS = 16
NW = NC * NS
CH = 32
NBUF = 6
VPR = D // 16
NSPLIT = 1
BH = B // NSPLIT


def _make_sc_gather(off):
    b_per_w = BH // NW
    n_ch = b_per_w // CH

    def body(uidx_hbm, iidx_hbm, ue_gmf, ie_gmf, ue_mlp, ie_mlp, wog_hbm,
             o_gd, o_um, o_im,
             uix, iix, wog, bufs, obufs, gsem, wsem):
        wid = lax.axis_index("s") * NC + lax.axis_index("c")
        base = off + wid * b_per_w
        obase = wid * b_per_w

        pltpu.sync_copy(wog_hbm, wog)
        for c in range(n_ch):
            pltpu.sync_copy(uidx_hbm.at[pl.ds(base + c * CH, CH)], uix.at[c])
            pltpu.sync_copy(iidx_hbm.at[pl.ds(base + c * CH, CH)], iix.at[c])

        def gather(c, s):
            bug, big, bum, bim = bufs[s]
            return [
                pltpu.async_copy(ue_gmf.at[uix.at[c]], bug, gsem),
                pltpu.async_copy(ie_gmf.at[iix.at[c]], big, gsem),
                pltpu.async_copy(ue_mlp.at[uix.at[c]], bum, gsem),
                pltpu.async_copy(ie_mlp.at[iix.at[c]], bim, gsem),
            ]

        lane = lax.iota(jnp.int32, 16)
        perms = [lane ^ (1 << k) for k in range(4)]
        shuf_dnums = lax.GatherDimensionNumbers(
            offset_dims=(), collapsed_slice_dims=(0,), start_index_map=(0,))

        def lane_sum(v):
            for p in perms:
                v = v + lax.gather(v, p[:, None], shuf_dnums, (1,),
                                   mode=lax.GatherScatterMode.PROMISE_IN_BOUNDS)
            return v

        def gmf_dot(s):
            bug, big = bufs[s][0], bufs[s][1]
            obuf = obufs[s]
            w = [wog[pl.ds(j * 16, 16)] for j in range(VPR)]

            def row(r, res):
                acc = bug[r, pl.ds(0, 16)] * big[r, pl.ds(0, 16)] * w[0]
                for j in range(1, VPR):
                    sl = pl.ds(j * 16, 16)
                    acc = acc + bug[r, sl] * big[r, sl] * w[j]
                total = lane_sum(acc)
                res = jnp.where(lane == lax.rem(r, 16), total, res)

                @pl.when(lax.rem(r, 16) == 15)
                def _():
                    obuf[pl.ds(r - 15, 16)] = res

                return res

            lax.fori_loop(0, CH, row, jnp.zeros((16,), jnp.float32))

        def writeback(c, s):
            _, _, bum, bim = bufs[s]
            rows = pl.ds(obase + c * CH, CH)
            return [
                pltpu.async_copy(obufs[s], o_gd.at[rows], wsem),
                pltpu.async_copy(bum, o_um.at[rows], wsem),
                pltpu.async_copy(bim, o_im.at[rows], wsem),
            ]

        g = {}
        wb = {}
        for c in range(min(2, n_ch)):
            g[c] = gather(c, c % NBUF)
        for c in range(n_ch):
            s = c % NBUF
            for dsc in g.pop(c):
                dsc.wait()
            gmf_dot(s)
            wb[s] = writeback(c, s)
            nc = c + 2
            if nc < n_ch:
                ns = nc % NBUF
                if ns in wb:
                    for dsc in wb.pop(ns):
                        dsc.wait()
                g[nc] = gather(nc, ns)
        for s in list(wb):
            for dsc in wb.pop(s):
                dsc.wait()

    return functools.partial(
        pl.kernel,
        mesh=plsc.VectorSubcoreMesh(core_axis_name="c", subcore_axis_name="s"),
        out_type=(jax.ShapeDtypeStruct((BH,), jnp.float32),
                  jax.ShapeDtypeStruct((BH, D), jnp.float32),
                  jax.ShapeDtypeStruct((BH, D), jnp.float32)),
        scratch_types=[
            pltpu.VMEM((n_ch, CH), jnp.int32),
            pltpu.VMEM((n_ch, CH), jnp.int32),
            pltpu.VMEM((D,), jnp.float32),
            tuple(tuple(pltpu.VMEM((CH, D), jnp.float32) for _ in range(4))
                  for _ in range(NBUF)),
            tuple(pltpu.VMEM((CH,), jnp.float32) for _ in range(NBUF)),
            pltpu.SemaphoreType.DMA,
            pltpu.SemaphoreType.DMA,
        ],
    )(body)


_sc_gather = [_make_sc_gather(h * BH) for h in range(NSPLIT)]


TC_BLK = 4096

_CONTRACT_01 = (((0,), (1,)), ((), ()))


def _tc_body(um_r, im_r, gd_r, w1_r, b1_r, wo_r, bo_r, out_r):
    w1 = w1_r[...]
    h_t = lax.dot_general(w1[0:D, :], um_r[...], _CONTRACT_01,
                          preferred_element_type=jnp.float32)
    h_t = h_t + lax.dot_general(w1[D:2 * D, :], im_r[...], _CONTRACT_01,
                                preferred_element_type=jnp.float32)
    h_t = jnp.maximum(h_t + b1_r[...].reshape(D, 1), 0.0)
    woh = wo_r[...][D:2 * D, :].T
    out = jnp.dot(woh, h_t, preferred_element_type=jnp.float32)
    out_r[...] = out + gd_r[...] + bo_r[...].reshape(1, 1)


def _tc_forward(um, im, gd_row, W1, b1, Wo, bo):
    grid = (BH // TC_BLK,)
    blk = lambda i: (i, 0)
    lane_blk = lambda i: (0, i)
    whole = lambda i: (0, 0)
    return pl.pallas_call(
        _tc_body,
        grid=grid,
        in_specs=[
            pl.BlockSpec((TC_BLK, D), blk),
            pl.BlockSpec((TC_BLK, D), blk),
            pl.BlockSpec((1, TC_BLK), lane_blk),
            pl.BlockSpec((2 * D, D), whole),
            pl.BlockSpec((D,), lambda i: (0,)),
            pl.BlockSpec((2 * D, 1), whole),
            pl.BlockSpec((1,), lambda i: (0,)),
        ],
        out_specs=pl.BlockSpec((1, TC_BLK), lane_blk),
        out_shape=jax.ShapeDtypeStruct((1, BH), jnp.float32),
    )(um, im, gd_row, W1, b1, Wo, bo)


def kernel(user_idx, item_idx, ue_gmf, ie_gmf, ue_mlp, ie_mlp, W1, b1, Wo, bo):
    wog = Wo[:D, 0]
    halves = []
    for h in range(NSPLIT):
        gd, um, im = _sc_gather[h](user_idx, item_idx, ue_gmf, ie_gmf,
                                   ue_mlp, ie_mlp, wog)
        halves.append((gd, um, im))
    outs = [_tc_forward(um, im, gd.reshape(1, BH), W1, b1, Wo, bo)
            for gd, um, im in halves]
    out = outs[0] if NSPLIT == 1 else jnp.concatenate(outs, axis=1)
    return out.reshape(B, 1)


# speedup vs baseline: 1.1705x; 1.1705x over previous
import functools

import jax
import jax.numpy as jnp
from jax import lax
from jax.experimental import pallas as pl
from jax.experimental.pallas import tpu as pltpu
from jax.experimental.pallas import tpu_sc as plsc

B = 16384
D = 128
NC = 2
NS = 16
NW = NC * NS
CH = 64
NBUF = 3
VPR = D // 16
NSPLIT = 1
BH = B // NSPLIT


def _make_sc_gather(off):
    b_per_w = BH // NW
    n_ch = b_per_w // CH

    def body(uidx_hbm, iidx_hbm, ue_gmf, ie_gmf, ue_mlp, ie_mlp, wog_hbm,
             o_gd, o_um, o_im,
             uix, iix, wog, bufs, obufs, gsem, wsem):
        wid = lax.axis_index("s") * NC + lax.axis_index("c")
        base = off + wid * b_per_w
        obase = wid * b_per_w

        pltpu.sync_copy(wog_hbm, wog)
        for c in range(n_ch):
            pltpu.sync_copy(uidx_hbm.at[pl.ds(base + c * CH, CH)], uix.at[c])
            pltpu.sync_copy(iidx_hbm.at[pl.ds(base + c * CH, CH)], iix.at[c])

        def gather(c, s):
            bug, big, bum, bim = bufs[s]
            return [
                pltpu.async_copy(ue_gmf.at[uix.at[c]], bug, gsem),
                pltpu.async_copy(ie_gmf.at[iix.at[c]], big, gsem),
                pltpu.async_copy(ue_mlp.at[uix.at[c]], bum, gsem),
                pltpu.async_copy(ie_mlp.at[iix.at[c]], bim, gsem),
            ]

        lane = lax.iota(jnp.int32, 16)
        perms = [lane ^ (1 << k) for k in range(4)]
        shuf_dnums = lax.GatherDimensionNumbers(
            offset_dims=(), collapsed_slice_dims=(0,), start_index_map=(0,))

        def lane_sum(v):
            for p in perms:
                v = v + lax.gather(v, p[:, None], shuf_dnums, (1,),
                                   mode=lax.GatherScatterMode.PROMISE_IN_BOUNDS)
            return v

        def gmf_dot(s):
            bug, big = bufs[s][0], bufs[s][1]
            obuf = obufs[s]
            w = [wog[pl.ds(j * 16, 16)] for j in range(VPR)]

            def row(r, res):
                acc = bug[r, pl.ds(0, 16)] * big[r, pl.ds(0, 16)] * w[0]
                for j in range(1, VPR):
                    sl = pl.ds(j * 16, 16)
                    acc = acc + bug[r, sl] * big[r, sl] * w[j]
                total = lane_sum(acc)
                res = jnp.where(lane == lax.rem(r, 16), total, res)

                @pl.when(lax.rem(r, 16) == 15)
                def _():
                    obuf[pl.ds(r - 15, 16)] = res

                return res

            lax.fori_loop(0, CH, row, jnp.zeros((16,), jnp.float32))

        def writeback(c, s):
            _, _, bum, bim = bufs[s]
            rows = pl.ds(obase + c * CH, CH)
            return [
                pltpu.async_copy(obufs[s], o_gd.at[rows], wsem),
                pltpu.async_copy(bum, o_um.at[rows], wsem),
                pltpu.async_copy(bim, o_im.at[rows], wsem),
            ]

        g = {}
        wb = {}
        for c in range(min(2, n_ch)):
            g[c] = gather(c, c % NBUF)
        for c in range(n_ch):
            s = c % NBUF
            for dsc in g.pop(c):
                dsc.wait()
            gmf_dot(s)
            wb[s] = writeback(c, s)
            nc = c + 2
            if nc < n_ch:
                ns = nc % NBUF
                if ns in wb:
                    for dsc in wb.pop(ns):
                        dsc.wait()
                g[nc] = gather(nc, ns)
        for s in list(wb):
            for dsc in wb.pop(s):
                dsc.wait()

    return functools.partial(
        pl.kernel,
        mesh=plsc.VectorSubcoreMesh(core_axis_name="c", subcore_axis_name="s"),
        out_type=(jax.ShapeDtypeStruct((BH,), jnp.float32),
                  jax.ShapeDtypeStruct((BH, D), jnp.float32),
                  jax.ShapeDtypeStruct((BH, D), jnp.float32)),
        scratch_types=[
            pltpu.VMEM((n_ch, CH), jnp.int32),
            pltpu.VMEM((n_ch, CH), jnp.int32),
            pltpu.VMEM((D,), jnp.float32),
            tuple(tuple(pltpu.VMEM((CH, D), jnp.float32) for _ in range(4))
                  for _ in range(NBUF)),
            tuple(pltpu.VMEM((CH,), jnp.float32) for _ in range(NBUF)),
            pltpu.SemaphoreType.DMA,
            pltpu.SemaphoreType.DMA,
        ],
    )(body)


_sc_gather = [_make_sc_gather(h * BH) for h in range(NSPLIT)]


TC_BLK = 4096

_CONTRACT_01 = (((0,), (1,)), ((), ()))


def _tc_body(um_r, im_r, gd_r, w1_r, b1_r, wo_r, bo_r, out_r):
    w1 = w1_r[...]
    h_t = lax.dot_general(w1[0:D, :], um_r[...], _CONTRACT_01,
                          preferred_element_type=jnp.float32)
    h_t = h_t + lax.dot_general(w1[D:2 * D, :], im_r[...], _CONTRACT_01,
                                preferred_element_type=jnp.float32)
    h_t = jnp.maximum(h_t + b1_r[...].reshape(D, 1), 0.0)
    woh = wo_r[...][D:2 * D, :].T
    out = jnp.dot(woh, h_t, preferred_element_type=jnp.float32)
    out_r[...] = out + gd_r[...] + bo_r[...].reshape(1, 1)


def _tc_forward(um, im, gd_row, W1, b1, Wo, bo):
    grid = (BH // TC_BLK,)
    blk = lambda i: (i, 0)
    lane_blk = lambda i: (0, i)
    whole = lambda i: (0, 0)
    return pl.pallas_call(
        _tc_body,
        grid=grid,
        in_specs=[
            pl.BlockSpec((TC_BLK, D), blk),
            pl.BlockSpec((TC_BLK, D), blk),
            pl.BlockSpec((1, TC_BLK), lane_blk),
            pl.BlockSpec((2 * D, D), whole),
            pl.BlockSpec((D,), lambda i: (0,)),
            pl.BlockSpec((2 * D, 1), whole),
            pl.BlockSpec((1,), lambda i: (0,)),
        ],
        out_specs=pl.BlockSpec((1, TC_BLK), lane_blk),
        out_shape=jax.ShapeDtypeStruct((1, BH), jnp.float32),
    )(um, im, gd_row, W1, b1, Wo, bo)


def kernel(user_idx, item_idx, ue_gmf, ie_gmf, ue_mlp, ie_mlp, W1, b1, Wo, bo):
    wog = Wo[:D, 0]
    halves = []
    for h in range(NSPLIT):
        gd, um, im = _sc_gather[h](user_idx, item_idx, ue_gmf, ie_gmf,
                                   ue_mlp, ie_mlp, wog)
        halves.append((gd, um, im))
    outs = [_tc_forward(um, im, gd.reshape(1, BH), W1, b1, Wo, bo)
            for gd, um, im in halves]
    out = outs[0] if NSPLIT == 1 else jnp.concatenate(outs, axis=1)
    return out.reshape(B, 1)
